# baseline (speedup 1.0000x reference)
import functools

import jax
import jax.numpy as jnp
from jax import lax
from jax.experimental import pallas as pl
from jax.experimental.pallas import tpu as pltpu
from jax.experimental.pallas import tpu_sc as plsc

B = 16384
N_ROWS = 1000000
MF_DIM = 8
D_UMLP = 16
D_IMLP = 16
D_KMLP = 32
L1 = 32
L2 = 16
CH = 4


def _sc_gather5(u_idx, i_idx, k_idx, t_umf, t_imf, t_umlp, t_imlp, t_kg,
                dmy):
    info = plsc.get_sparse_core_info()
    nw = info.num_cores * info.num_subcores
    bw = B // nw

    mesh = plsc.VectorSubcoreMesh(core_axis_name="c", subcore_axis_name="s")
    out_type = (
        jax.ShapeDtypeStruct((MF_DIM, B), jnp.float32),
        jax.ShapeDtypeStruct((MF_DIM, B), jnp.float32),
        jax.ShapeDtypeStruct((D_UMLP, B), jnp.float32),
        jax.ShapeDtypeStruct((D_IMLP, B), jnp.float32),
        jax.ShapeDtypeStruct((D_KMLP, B), jnp.float32),
    )
    scratch = [
        pltpu.VMEM((bw + 16,), jnp.int32),
        pltpu.VMEM((bw + 16,), jnp.int32),
        pltpu.VMEM((bw + 16,), jnp.int32),
        pltpu.VMEM((2, CH, 8, 128), jnp.float32),
        pltpu.VMEM((2, CH, 8, 128), jnp.float32),
        pltpu.VMEM((2, 2 * CH, 8, 128), jnp.float32),
        pltpu.VMEM((2, 2 * CH, 8, 128), jnp.float32),
        pltpu.VMEM((2, 4 * CH, 8, 128), jnp.float32),
        pltpu.VMEM((MF_DIM, bw), jnp.float32),
        pltpu.VMEM((MF_DIM, bw), jnp.float32),
        pltpu.VMEM((D_UMLP, bw), jnp.float32),
        pltpu.VMEM((D_IMLP, bw), jnp.float32),
        pltpu.VMEM((D_KMLP, bw), jnp.float32),
    ] + [pltpu.SemaphoreType.DMA] * 10

    @functools.partial(
        pl.kernel, out_type=out_type, mesh=mesh, scratch_types=scratch,
        compiler_params=pltpu.CompilerParams(needs_layout_passes=False),
    )
    def gather_kernel(u_h, i_h, k_h, tum_h, tim_h, tuml_h, timl_h, tkg_h,
                      dmy_h, o_um, o_im, o_uml, o_iml, o_kg,
                      xu, xi, xk, g_um, g_im, g_uml, g_iml, g_kg,
                      c_um, c_im, c_uml, c_iml, c_kg, *sems):
        wid = lax.axis_index("s") * info.num_cores + lax.axis_index("c")
        base = wid * bw
        pltpu.sync_copy(u_h.at[pl.ds(base, bw)], xu.at[pl.ds(0, bw)])
        pltpu.sync_copy(i_h.at[pl.ds(base, bw)], xi.at[pl.ds(0, bw)])
        pltpu.sync_copy(k_h.at[pl.ds(base, bw)], xk.at[pl.ds(0, bw)])

        it = lax.iota(jnp.int32, 16)
        d8 = it & 7
        m8 = it < 8
        sel = it >> 3
        nph = bw // CH

        def fire(p, b):
            vu = xu[pl.ds(p * CH, 16)]
            vi = xi[pl.ds(p * CH, 16)]
            vk = xk[pl.ds(p * CH, 16)]
            cbu = vu & (-128)
            cbi = vi & (-128)
            cbk = vk & (-128)
            s0, s1, s2, s3, s4 = sems[5 * b:5 * b + 5]
            for t in range(CH):
                cu = pl.multiple_of(cbu[t], 128)
                ci = pl.multiple_of(cbi[t], 128)
                ck = pl.multiple_of(cbk[t], 128)
                pltpu.async_copy(
                    tum_h.at[0, :, pl.ds(cu, 128)], g_um.at[b, t], s0)
                pltpu.async_copy(
                    tim_h.at[0, :, pl.ds(ci, 128)], g_im.at[b, t], s1)
                pltpu.async_copy(
                    tuml_h.at[:, :, pl.ds(cu, 128)],
                    g_uml.at[b, pl.ds(2 * t, 2)], s2)
                pltpu.async_copy(
                    timl_h.at[:, :, pl.ds(ci, 128)],
                    g_iml.at[b, pl.ds(2 * t, 2)], s3)
                pltpu.async_copy(
                    tkg_h.at[:, :, pl.ds(ck, 128)],
                    g_kg.at[b, pl.ds(4 * t, 4)], s4)

        def drain(b):
            s0, s1, s2, s3, s4 = sems[5 * b:5 * b + 5]
            pltpu.make_async_copy(
                dmy_h.at[pl.ds(0, CH)], g_um.at[b], s0).wait()
            pltpu.make_async_copy(
                dmy_h.at[pl.ds(0, CH)], g_im.at[b], s1).wait()
            pltpu.make_async_copy(
                dmy_h.at[pl.ds(0, 2 * CH)], g_uml.at[b], s2).wait()
            pltpu.make_async_copy(
                dmy_h.at[pl.ds(0, 2 * CH)], g_iml.at[b], s3).wait()
            pltpu.make_async_copy(
                dmy_h.at[pl.ds(0, 4 * CH)], g_kg.at[b], s4).wait()

        def select(p, b):
            vu = xu[pl.ds(p * CH, 16)]
            vi = xi[pl.ds(p * CH, 16)]
            vk = xk[pl.ds(p * CH, 16)]
            lnu = vu & 127
            lni = vi & 127
            lnk = vk & 127
            bv = jnp.zeros((16,), jnp.int32) + b
            for t in range(CH):
                col = p * CH + t
                colv = col + jnp.zeros((16,), jnp.int32)
                tv = jnp.zeros((16,), jnp.int32) + t

                lu = lnu[t] + jnp.zeros((16,), jnp.int32)
                vals = plsc.load_gather(g_um, [bv, tv, d8, lu], mask=m8)
                plsc.store_scatter(c_um, [d8, colv], vals, mask=m8)

                li = lni[t] + jnp.zeros((16,), jnp.int32)
                vals = plsc.load_gather(g_im, [bv, tv, d8, li], mask=m8)
                plsc.store_scatter(c_im, [d8, colv], vals, mask=m8)

                vals = plsc.load_gather(g_uml, [bv, 2 * tv + sel, d8, lu])
                plsc.store_scatter(c_uml, [it, colv], vals)
                vals = plsc.load_gather(g_iml, [bv, 2 * tv + sel, d8, li])
                plsc.store_scatter(c_iml, [it, colv], vals)

                lk = lnk[t] + jnp.zeros((16,), jnp.int32)
                vals = plsc.load_gather(g_kg, [bv, 4 * tv + sel, d8, lk])
                plsc.store_scatter(c_kg, [it, colv], vals)
                vals = plsc.load_gather(g_kg, [bv, 4 * tv + 2 + sel, d8, lk])
                plsc.store_scatter(c_kg, [16 + it, colv], vals)

        fire(0, 0)

        def pair(q, _):
            p0 = 2 * q
            fire(p0 + 1, 1)
            drain(0)
            select(p0, 0)
            fire(lax.min(p0 + 2, nph - 1), 0)
            drain(1)
            select(p0 + 1, 1)
            return 0

        lax.fori_loop(0, nph // 2, pair, 0)
        drain(0)

        pltpu.sync_copy(c_um, o_um.at[:, pl.ds(base, bw)])
        pltpu.sync_copy(c_im, o_im.at[:, pl.ds(base, bw)])
        pltpu.sync_copy(c_uml, o_uml.at[:, pl.ds(base, bw)])
        pltpu.sync_copy(c_iml, o_iml.at[:, pl.ds(base, bw)])
        pltpu.sync_copy(c_kg, o_kg.at[:, pl.ds(base, bw)])

    return gather_kernel(u_idx, i_idx, k_idx, t_umf, t_imf, t_umlp, t_imlp,
                         t_kg, dmy)


def _mlp_body(um, im, uml, iml, kg, w1u, w1i, w1k, b1, w2, b2, wpm, wph,
              bp, out):
    f32 = jnp.float32
    h1 = (
        jnp.dot(w1u[...], uml[...], preferred_element_type=f32)
        + jnp.dot(w1i[...], iml[...], preferred_element_type=f32)
        + jnp.dot(w1k[...], kg[...], preferred_element_type=f32)
        + b1[...]
    )
    h1 = jnp.maximum(h1, 0.0)
    h2 = jnp.maximum(
        jnp.dot(w2[...], h1, preferred_element_type=f32) + b2[...], 0.0)
    mf = um[...] * im[...]
    logits = (
        jnp.dot(wpm[...], mf, preferred_element_type=f32)
        + jnp.dot(wph[...], h2, preferred_element_type=f32)
        + bp[...]
    )
    out[...] = jax.nn.sigmoid(logits[0, :])


def _tc_mlp(um, im, uml, iml, kg, w1u, w1i, w1k, b1, w2, b2, wpm, wph, bp):
    nb = 2048
    grid = (B // nb,)

    def col_spec(d):
        return pl.BlockSpec((d, nb), lambda i: (0, i))

    def full_spec(shape):
        return pl.BlockSpec(shape, lambda i: tuple(0 for _ in shape))

    return pl.pallas_call(
        _mlp_body,
        grid=grid,
        in_specs=[
            col_spec(MF_DIM), col_spec(MF_DIM),
            col_spec(D_UMLP), col_spec(D_IMLP), col_spec(D_KMLP),
            full_spec((L1, D_UMLP)), full_spec((L1, D_IMLP)),
            full_spec((L1, D_KMLP)), full_spec((L1, 1)),
            full_spec((L2, L1)), full_spec((L2, 1)),
            full_spec((1, MF_DIM)), full_spec((1, L2)), full_spec((1, 1)),
        ],
        out_specs=pl.BlockSpec((nb,), lambda i: (i,)),
        out_shape=jax.ShapeDtypeStruct((B,), jnp.float32),
    )(um, im, uml, iml, kg, w1u, w1i, w1k, b1, w2, b2, wpm, wph, bp)


def kernel(user_indices, item_indices, kg_indices, E_user_mf, E_item_mf,
           E_user_mlp, E_item_mlp, E_kg_mlp, W1, b1, W2, b2, Wp, bp):
    t_umf = E_user_mf.T.reshape(1, MF_DIM, N_ROWS)
    t_imf = E_item_mf.T.reshape(1, MF_DIM, N_ROWS)
    t_umlp = E_user_mlp.T.reshape(D_UMLP // 8, 8, N_ROWS)
    t_imlp = E_item_mlp.T.reshape(D_IMLP // 8, 8, N_ROWS)
    t_kg = E_kg_mlp.T.reshape(D_KMLP // 8, 8, N_ROWS)
    dmy = jnp.zeros((4 * CH, 8, 128), jnp.float32)

    um, im, uml, iml, kg = _sc_gather5(
        user_indices, item_indices, kg_indices,
        t_umf, t_imf, t_umlp, t_imlp, t_kg, dmy)

    w1t = W1.T
    w1u = w1t[:, 0:D_UMLP]
    w1i = w1t[:, D_UMLP:D_UMLP + D_IMLP]
    w1k = w1t[:, D_UMLP + D_IMLP:]
    wpt = Wp.T
    return _tc_mlp(um, im, uml, iml, kg, w1u, w1i, w1k,
                   b1.reshape(L1, 1), W2.T, b2.reshape(L2, 1),
                   wpt[:, 0:MF_DIM], wpt[:, MF_DIM:], bp.reshape(1, 1))
